# baseline (speedup 1.0000x reference)
import functools

import jax
import jax.numpy as jnp
from jax import lax
from jax.experimental import pallas as pl
from jax.experimental.pallas import tpu as pltpu
from jax.experimental.pallas import tpu_sc as plsc

_C_IN = 1000
_D = 64
_BATCH = 16384
_HIST = 200

_NC = 2
_NS = 16
_NW = _NC * _NS

_B = _BATCH * _HIST
_ROWS_PER_W = _B // _NW
_GATHER = 128
_KPG = 2
_GROUP = _GATHER * _KPG
_NGROUPS = _ROWS_PER_W // _GROUP
_NBUF = 4


def _sc_body(x_hbm, w_hbm, out_hbm, idx_v, rows_v, sems):
    wid = lax.axis_index("s") * _NC + lax.axis_index("c")
    sem_i = [sems.at[0, b] for b in range(_NBUF)]
    sem_g = [sems.at[1, b] for b in range(_NBUF)]
    sem_o = [sems.at[2, b] for b in range(_NBUF)]

    def start_idx(buf, g):
        pltpu.make_async_copy(x_hbm.at[wid, g], idx_v.at[buf], sem_i[buf]).start()

    def wait_idx(buf):
        pltpu.make_async_copy(x_hbm.at[wid, 0], idx_v.at[buf], sem_i[buf]).wait()

    def start_gathers(buf):
        for k in range(_KPG):
            pltpu.make_async_copy(
                w_hbm.at[idx_v.at[buf, k]],
                rows_v.at[buf, pl.ds(k * _GATHER, _GATHER)],
                sem_g[buf],
            ).start()

    def wait_gathers(buf):
        for k in range(_KPG):
            pltpu.make_async_copy(
                w_hbm.at[idx_v.at[buf, k]],
                rows_v.at[buf, pl.ds(k * _GATHER, _GATHER)],
                sem_g[buf],
            ).wait()

    def start_out(buf, g):
        pltpu.make_async_copy(rows_v.at[buf], out_hbm.at[wid, g], sem_o[buf]).start()

    def wait_out(buf):
        pltpu.make_async_copy(rows_v.at[buf], out_hbm.at[wid, 0], sem_o[buf]).wait()

    for b in range(_NBUF):
        start_idx(b, b)
    wait_idx(0)
    start_gathers(0)
    for g in range(1, _NBUF):
        wait_idx(g)
        start_gathers(g)
        b1 = g - 1
        wait_gathers(b1)
        start_idx(b1, b1 + _NBUF)
        start_out(b1, b1)

    def quad(q, carry):
        g0 = _NBUF * q
        for r in range(_NBUF):
            b = r
            g = g0 + r
            wait_out(b)
            wait_idx(b)
            start_gathers(b)
            b1 = (r - 1) % _NBUF
            wait_gathers(b1)
            start_idx(b1, lax.rem(g - 1 + _NBUF, _NGROUPS))
            start_out(b1, g - 1)
        return carry

    lax.fori_loop(1, _NGROUPS // _NBUF, quad, 0, unroll=False)

    last_b = (_NGROUPS - 1) % _NBUF
    wait_gathers(last_b)
    start_out(last_b, _NGROUPS - 1)
    for b in range(_NBUF):
        wait_out(b)
    for b in range(_NBUF - 1):
        wait_idx(b)


@jax.jit
def _lookup(x32, w):
    mesh = plsc.VectorSubcoreMesh(
        core_axis_name="c", subcore_axis_name="s",
        num_cores=_NC, num_subcores=_NS,
    )
    run = pl.kernel(
        _sc_body,
        out_type=jax.ShapeDtypeStruct((_NW, _NGROUPS, _GROUP, _D), jnp.bfloat16),
        mesh=mesh,
        scratch_types=[
            pltpu.VMEM((_NBUF, _KPG, _GATHER), jnp.int32),
            pltpu.VMEM((_NBUF, _GROUP, _D), jnp.bfloat16),
            pltpu.SemaphoreType.DMA((3, _NBUF)),
        ],
        compiler_params=pltpu.CompilerParams(use_tc_tiling_on_sc=False),
    )
    return run(x32, w)


def kernel(x, W):
    x32 = x.reshape(-1).astype(jnp.int32).reshape(_NW, _NGROUPS, _KPG, _GATHER)
    out = _lookup(x32, W.astype(jnp.bfloat16))
    return lax.stop_gradient(
        out.astype(jnp.float32).reshape(_BATCH, _HIST, _D))


# speedup vs baseline: 1.5629x; 1.5629x over previous
import functools

import jax
import jax.numpy as jnp
from jax import lax
from jax.experimental import pallas as pl
from jax.experimental.pallas import tpu as pltpu
from jax.experimental.pallas import tpu_sc as plsc

_C_IN = 1000
_D = 64
_BATCH = 16384
_HIST = 200

_NC = 2
_NS = 16
_NW = _NC * _NS
_L = 16

_B = _BATCH * _HIST
_ROWS_PER_W = _B // _NW
_GATHER = 80
_KPG = 2
_GROUP = _GATHER * _KPG
_NGROUPS = _ROWS_PER_W // _GROUP
_NBUF = 4
_TBL = (_C_IN + 1) * _D


def _sc_body(x_hbm, w_hbm, out_hbm, table_v, idx_v, rows_v, sems):
    wid = lax.axis_index("s") * _NC + lax.axis_index("c")
    sem_t = sems.at[0, 0]
    sem_i = [sems.at[1, b] for b in range(_NBUF)]
    sem_g = [sems.at[2, b] for b in range(_NBUF)]
    sem_o = [sems.at[3, b] for b in range(_NBUF)]

    def start_idx(buf, g):
        pltpu.make_async_copy(x_hbm.at[wid, g], idx_v.at[buf], sem_i[buf]).start()

    def wait_idx(buf):
        pltpu.make_async_copy(x_hbm.at[wid, 0], idx_v.at[buf], sem_i[buf]).wait()

    def start_gathers(buf):
        for k in range(_KPG):
            pltpu.make_async_copy(
                w_hbm.at[idx_v.at[buf, k]],
                rows_v.at[buf, pl.ds(k * _GATHER, _GATHER)],
                sem_g[buf],
            ).start()

    def wait_gathers(buf):
        for k in range(_KPG):
            pltpu.make_async_copy(
                w_hbm.at[idx_v.at[buf, k]],
                rows_v.at[buf, pl.ds(k * _GATHER, _GATHER)],
                sem_g[buf],
            ).wait()

    def start_out(buf, g):
        pltpu.make_async_copy(rows_v.at[buf], out_hbm.at[wid, g], sem_o[buf]).start()

    def wait_out(buf):
        pltpu.make_async_copy(rows_v.at[buf], out_hbm.at[wid, 0], sem_o[buf]).wait()

    pltpu.make_async_copy(w_hbm, table_v, sem_t).start()
    for b in range(_NBUF):
        start_idx(b, b)
    pltpu.make_async_copy(w_hbm, table_v, sem_t).wait()

    def compute(buf):
        rows = rows_v.at[buf]

        def tile(t, carry):
            for k in range(_KPG):
                bases = idx_v[buf, k, pl.ds(t * _L, _L)]
                for u in range(_L):
                    base = bases[u]
                    r = k * _GATHER + t * _L + u
                    for c in range(0, _D, _L):
                        rows[r, pl.ds(c, _L)] = table_v[base, pl.ds(c, _L)]
            return carry

        lax.fori_loop(0, _GATHER // _L, tile, 0, unroll=False)

    def quad(g0, first):
        for b in (0, 1):
            if not first:
                wait_out(b)
            wait_idx(b)
            start_gathers(b)
        for b in (2, 3):
            if not first:
                wait_out(b)
            wait_idx(b)
            compute(b)
            start_idx(b, lax.rem(g0 + b + _NBUF, _NGROUPS))
            start_out(b, g0 + b)
            b1 = b - 2
            wait_gathers(b1)
            start_idx(b1, lax.rem(g0 + b1 + _NBUF, _NGROUPS))
            start_out(b1, g0 + b1)

    quad(0, True)

    def body(q, carry):
        quad(_NBUF * q, False)
        return carry

    lax.fori_loop(1, _NGROUPS // _NBUF, body, 0, unroll=False)

    for b in range(_NBUF):
        wait_out(b)
        wait_idx(b)


@jax.jit
def _lookup(x32, w):
    mesh = plsc.VectorSubcoreMesh(
        core_axis_name="c", subcore_axis_name="s",
        num_cores=_NC, num_subcores=_NS,
    )
    run = pl.kernel(
        _sc_body,
        out_type=jax.ShapeDtypeStruct((_NW, _NGROUPS, _GROUP, _D), jnp.float32),
        mesh=mesh,
        scratch_types=[
            pltpu.VMEM((_C_IN + 1, _D), jnp.float32),
            pltpu.VMEM((_NBUF, _KPG, _GATHER), jnp.int32),
            pltpu.VMEM((_NBUF, _GROUP, _D), jnp.float32),
            pltpu.SemaphoreType.DMA((4, _NBUF)),
        ],
        compiler_params=pltpu.CompilerParams(
            use_tc_tiling_on_sc=False, needs_layout_passes=False),
    )
    return run(x32, w)


def kernel(x, W):
    x32 = x.reshape(-1).astype(jnp.int32).reshape(_NW, _NGROUPS, _KPG, _GATHER)
    out = _lookup(x32, W)
    return lax.stop_gradient(out.reshape(_BATCH, _HIST, _D))
